# Initial kernel scaffold; baseline (speedup 1.0000x reference)
import jax
import jax.numpy as jnp
from jax.experimental import pallas as pl


def kernel(ref_bxyz, ref_feat, query_bxyz, e_ref, e_query, W_pos, b_pos, W_mlp, b_mlp, bn_gamma, bn_beta):
    raise NotImplementedError("write your pallas kernel here")


# speedup vs baseline: 12.3279x; 12.3279x over previous
import functools

import jax
import jax.numpy as jnp
from jax import lax
from jax.experimental import pallas as pl
from jax.experimental.pallas import tpu as pltpu
from jax.experimental.pallas import tpu_sc as plsc

N = 10000
E = 320000
M = 10000
C = 128
D = 144
NC = 2
NS = 16
NW = NC * NS
EPW = E // NW
K = 80
NCHUNK = EPW // K
RPS = M // NS


def _sc_body(t_hbm, eref_hbm, eq_hbm, z_hbm, out_hbm,
             eref_v, eq_v, rows_v, acc_sh, sem):
    c = lax.axis_index("c")
    s = lax.axis_index("s")
    wid = c * NS + s
    pltpu.sync_copy(z_hbm, acc_sh.at[pl.ds(s * RPS, RPS)])
    pltpu.sync_copy(eref_hbm.at[wid], eref_v)
    pltpu.sync_copy(eq_hbm.at[wid], eq_v)
    plsc.subcore_barrier()

    def chunk(i, carry):
        pltpu.async_copy(t_hbm.at[eref_v.at[i]], rows_v, sem).wait()
        pltpu.sync_copy(rows_v, acc_sh.at[eq_v.at[i]], add=True)
        return carry

    lax.fori_loop(0, NCHUNK, chunk, 0)
    plsc.subcore_barrier()
    pltpu.sync_copy(acc_sh.at[pl.ds(s * RPS, RPS)],
                    out_hbm.at[c].at[pl.ds(s * RPS, RPS)])


def _sc_scatter(table, eref_w, eq_w, zrows):
    mesh = plsc.VectorSubcoreMesh(
        core_axis_name="c", subcore_axis_name="s", num_cores=NC, num_subcores=NS)
    return pl.kernel(
        _sc_body,
        out_type=jax.ShapeDtypeStruct((NC, M, D), jnp.float32),
        mesh=mesh,
        scratch_types=[
            pltpu.VMEM((NCHUNK, K), jnp.int32),
            pltpu.VMEM((NCHUNK, K), jnp.int32),
            pltpu.VMEM((K, D), jnp.float32),
            pltpu.VMEM_SHARED((M, D), jnp.float32),
            pltpu.SemaphoreType.DMA,
        ],
        compiler_params=pltpu.CompilerParams(use_tc_tiling_on_sc=False),
    )(table, eref_w, eq_w, zrows)


def _dense_body(f0, f1, t0, t1, qb, wm, wt, sel, wq, bias, gamma, beta, out):
    dot = functools.partial(jnp.dot, preferred_element_type=jnp.float32)
    sfeat = f0[...] + f1[...]
    tail = t0[...] + t1[...]
    counts = dot(tail, sel[...])
    numer = (dot(sfeat, wm[...])
             + dot(tail, wt[...])
             + counts * (bias[...] - dot(qb[...], wq[...])))
    qf = numer / jnp.maximum(counts, 1.0)
    mean = jnp.mean(qf, axis=0, keepdims=True)
    dev = qf - mean
    var = jnp.mean(dev * dev, axis=0, keepdims=True)
    out[...] = jnp.maximum(
        dev * lax.rsqrt(var + 1e-5) * gamma[...] + beta[...], 0.0)


@jax.jit
def kernel(ref_bxyz, ref_feat, query_bxyz, e_ref, e_query,
           W_pos, b_pos, W_mlp, b_mlp, bn_gamma, bn_beta):
    f32 = jnp.float32
    table = jnp.concatenate(
        [ref_feat,
         ref_bxyz[:, 1:4],
         jnp.ones((N, 1), f32),
         jnp.zeros((N, D - C - 4), f32)], axis=1)
    eref_w = e_ref.reshape(NW, NCHUNK, K)
    eq_w = e_query.reshape(NW, NCHUNK, K)
    zrows = jnp.zeros((RPS, D), f32)

    partials = _sc_scatter(table, eref_w, eq_w, zrows)

    f0 = partials[0, :, :C]
    f1 = partials[1, :, :C]
    t0 = partials[0, :, C:C + 16]
    t1 = partials[1, :, C:C + 16]
    wt = jnp.zeros((16, C), f32).at[0:3, :].set(W_pos.T)
    sel = jnp.zeros((16, C), f32).at[3, :].set(1.0)
    wq = jnp.concatenate([jnp.zeros((1, C), f32), W_pos.T], axis=0)
    bias = (b_mlp + b_pos).reshape(1, C)

    return pl.pallas_call(
        _dense_body,
        out_shape=jax.ShapeDtypeStruct((M, C), f32),
    )(f0, f1, t0, t1, query_bxyz, W_mlp.T, wt, sel, wq, bias,
      bn_gamma.reshape(1, C), bn_beta.reshape(1, C))


# speedup vs baseline: 15.4385x; 1.2523x over previous
import functools

import jax
import jax.numpy as jnp
from jax import lax
from jax.experimental import pallas as pl
from jax.experimental.pallas import tpu as pltpu
from jax.experimental.pallas import tpu_sc as plsc

N = 10000
E = 320000
M = 10000
C = 128
D = 144
NC = 2
NS = 16
NW = NC * NS
EPW = E // NW
K = 125
NCHUNK = EPW // K
RPS = M // NS


def _sc_body(t_hbm, comb_hbm, z_hbm, out_hbm, idx_v, rows_v, acc_sh, sem):
    c = lax.axis_index("c")
    s = lax.axis_index("s")
    wid = c * NS + s
    pltpu.sync_copy(z_hbm, acc_sh.at[pl.ds(s * RPS, RPS)])
    plsc.subcore_barrier()

    pltpu.sync_copy(comb_hbm.at[wid, 0], idx_v.at[0])
    pltpu.async_copy(t_hbm.at[idx_v.at[0, 0]], rows_v.at[0], sem)

    def chunk(i, carry):
        b = lax.rem(i, 2)

        @pl.when(i + 1 < NCHUNK)
        def _prefetch():
            pltpu.sync_copy(comb_hbm.at[wid, i + 1], idx_v.at[1 - b])
            pltpu.async_copy(t_hbm.at[idx_v.at[1 - b, 0]], rows_v.at[1 - b],
                             sem)

        pltpu.make_async_copy(t_hbm.at[idx_v.at[b, 0]], rows_v.at[b],
                              sem).wait()
        pltpu.sync_copy(rows_v.at[b], acc_sh.at[idx_v.at[b, 1]], add=True)
        return carry

    lax.fori_loop(0, NCHUNK, chunk, 0)
    plsc.subcore_barrier()
    pltpu.sync_copy(acc_sh.at[pl.ds(s * RPS, RPS)],
                    out_hbm.at[c].at[pl.ds(s * RPS, RPS)])


def _sc_scatter(table, comb, zrows):
    mesh = plsc.VectorSubcoreMesh(
        core_axis_name="c", subcore_axis_name="s", num_cores=NC, num_subcores=NS)
    return pl.kernel(
        _sc_body,
        out_type=jax.ShapeDtypeStruct((NC, M, D), jnp.float32),
        mesh=mesh,
        scratch_types=[
            pltpu.VMEM((2, 2, K), jnp.int32),
            pltpu.VMEM((2, K, D), jnp.float32),
            pltpu.VMEM_SHARED((M, D), jnp.float32),
            pltpu.SemaphoreType.DMA,
        ],
        compiler_params=pltpu.CompilerParams(use_tc_tiling_on_sc=False),
    )(table, comb, zrows)


def _dense_body(f0, f1, t0, t1, qb, wm, wt, sel, wq, bias, gamma, beta, out):
    dot = functools.partial(jnp.dot, preferred_element_type=jnp.float32)
    sfeat = f0[...] + f1[...]
    tail = t0[...] + t1[...]
    counts = dot(tail, sel[...])
    numer = (dot(sfeat, wm[...])
             + dot(tail, wt[...])
             + counts * (bias[...] - dot(qb[...], wq[...])))
    qf = numer / jnp.maximum(counts, 1.0)
    mean = jnp.mean(qf, axis=0, keepdims=True)
    dev = qf - mean
    var = jnp.mean(dev * dev, axis=0, keepdims=True)
    out[...] = jnp.maximum(
        dev * lax.rsqrt(var + 1e-5) * gamma[...] + beta[...], 0.0)


@jax.jit
def kernel(ref_bxyz, ref_feat, query_bxyz, e_ref, e_query,
           W_pos, b_pos, W_mlp, b_mlp, bn_gamma, bn_beta):
    f32 = jnp.float32
    table = jnp.concatenate(
        [ref_feat,
         ref_bxyz[:, 1:4],
         jnp.ones((N, 1), f32),
         jnp.zeros((N, D - C - 4), f32)], axis=1)
    comb = jnp.stack([e_ref.reshape(NW, NCHUNK, K),
                      e_query.reshape(NW, NCHUNK, K)], axis=2)
    zrows = jnp.zeros((RPS, D), f32)

    partials = _sc_scatter(table, comb, zrows)

    f0 = partials[0, :, :C]
    f1 = partials[1, :, :C]
    t0 = partials[0, :, C:C + 16]
    t1 = partials[1, :, C:C + 16]
    wt = jnp.zeros((16, C), f32).at[0:3, :].set(W_pos.T)
    sel = jnp.zeros((16, C), f32).at[3, :].set(1.0)
    wq = jnp.concatenate([jnp.zeros((1, C), f32), W_pos.T], axis=0)
    bias = (b_mlp + b_pos).reshape(1, C)

    return pl.pallas_call(
        _dense_body,
        out_shape=jax.ShapeDtypeStruct((M, C), f32),
    )(f0, f1, t0, t1, query_bxyz, W_mlp.T, wt, sel, wq, bias,
      bn_gamma.reshape(1, C), bn_beta.reshape(1, C))


# speedup vs baseline: 18.3741x; 1.1901x over previous
import functools

import jax
import jax.numpy as jnp
from jax import lax
from jax.experimental import pallas as pl
from jax.experimental.pallas import tpu as pltpu
from jax.experimental.pallas import tpu_sc as plsc

N = 10000
E = 320000
M = 10000
C = 128
D = 144
NC = 2
NS = 16
NW = NC * NS
EPW = E // NW
K = 125
NCHUNK = EPW // K
RPS = M // NS


def _sc_body(t_hbm, comb_hbm, z_hbm, out_hbm, idx_v, rows_v, acc_sh, sem,
             isem):
    c = lax.axis_index("c")
    s = lax.axis_index("s")
    wid = c * NS + s
    pltpu.sync_copy(z_hbm, acc_sh.at[pl.ds(s * RPS, RPS)])
    plsc.subcore_barrier()

    pltpu.sync_copy(comb_hbm.at[wid, 0], idx_v.at[0])
    pltpu.async_copy(comb_hbm.at[wid, 1], idx_v.at[1], isem)
    pltpu.async_copy(t_hbm.at[idx_v.at[0, 0]], rows_v.at[0], sem)

    def chunk(i, carry):
        b3 = lax.rem(i, 3)
        b2 = lax.rem(i, 2)
        n3 = lax.rem(i + 1, 3)

        @pl.when(i + 1 < NCHUNK)
        def _fire_gather():
            pltpu.make_async_copy(comb_hbm.at[wid, i + 1], idx_v.at[n3],
                                  isem).wait()
            pltpu.async_copy(t_hbm.at[idx_v.at[n3, 0]], rows_v.at[1 - b2],
                             sem)

        @pl.when(i + 2 < NCHUNK)
        def _fire_idx():
            pltpu.async_copy(comb_hbm.at[wid, i + 2],
                             idx_v.at[lax.rem(i + 2, 3)], isem)

        pltpu.make_async_copy(t_hbm.at[idx_v.at[b3, 0]], rows_v.at[b2],
                              sem).wait()
        pltpu.sync_copy(rows_v.at[b2], acc_sh.at[idx_v.at[b3, 1]], add=True)
        return carry

    lax.fori_loop(0, NCHUNK, chunk, 0)
    plsc.subcore_barrier()
    pltpu.sync_copy(acc_sh.at[pl.ds(s * RPS, RPS)],
                    out_hbm.at[c].at[pl.ds(s * RPS, RPS)])


def _sc_scatter(table, comb, zrows):
    mesh = plsc.VectorSubcoreMesh(
        core_axis_name="c", subcore_axis_name="s", num_cores=NC, num_subcores=NS)
    return pl.kernel(
        _sc_body,
        out_type=jax.ShapeDtypeStruct((NC, M, D), jnp.float32),
        mesh=mesh,
        scratch_types=[
            pltpu.VMEM((3, 2, K), jnp.int32),
            pltpu.VMEM((2, K, D), jnp.float32),
            pltpu.VMEM_SHARED((M, D), jnp.float32),
            pltpu.SemaphoreType.DMA,
            pltpu.SemaphoreType.DMA,
        ],
        compiler_params=pltpu.CompilerParams(use_tc_tiling_on_sc=False),
    )(table, comb, zrows)


def _dense_body(pref, qb, wcomb, sel, wq, gamma, beta, out):
    dot = functools.partial(jnp.dot, preferred_element_type=jnp.float32)
    acc = pref[0] + pref[1]
    counts = dot(acc, sel[...])
    numer = dot(acc, wcomb[...]) - counts * dot(qb[...], wq[...])
    qf = numer / jnp.maximum(counts, 1.0)
    mean = jnp.mean(qf, axis=0, keepdims=True)
    dev = qf - mean
    var = jnp.mean(dev * dev, axis=0, keepdims=True)
    out[...] = jnp.maximum(
        dev * lax.rsqrt(var + 1e-5) * gamma[...] + beta[...], 0.0)


@jax.jit
def kernel(ref_bxyz, ref_feat, query_bxyz, e_ref, e_query,
           W_pos, b_pos, W_mlp, b_mlp, bn_gamma, bn_beta):
    f32 = jnp.float32
    table = jnp.concatenate(
        [ref_feat,
         ref_bxyz[:, 1:4],
         jnp.ones((N, 1), f32),
         jnp.zeros((N, D - C - 4), f32)], axis=1)
    comb = jnp.stack([e_ref.reshape(NW, NCHUNK, K),
                      e_query.reshape(NW, NCHUNK, K)], axis=2)
    zrows = jnp.zeros((RPS, D), f32)

    partials = _sc_scatter(table, comb, zrows)

    wcomb = jnp.concatenate(
        [W_mlp.T, W_pos.T, (b_mlp + b_pos)[None, :],
         jnp.zeros((D - C - 4, C), f32)], axis=0)
    sel = jnp.zeros((D, C), f32).at[C + 3, :].set(1.0)
    wq = jnp.concatenate([jnp.zeros((1, C), f32), W_pos.T], axis=0)

    return pl.pallas_call(
        _dense_body,
        out_shape=jax.ShapeDtypeStruct((M, C), f32),
    )(partials, query_bxyz, wcomb, sel, wq,
      bn_gamma.reshape(1, C), bn_beta.reshape(1, C))
